# baseline (speedup 1.0000x reference)
import functools

import jax
import jax.numpy as jnp
from jax import lax
from jax.experimental import pallas as pl
from jax.experimental.pallas import tpu as pltpu
from jax.experimental.pallas import tpu_sc as plsc

_SPK_NUM = 1000000
_EMB_DIM = 64
_BATCH = 16384

_NC = 2
_NS = 16
_L = 16
_NW = _NC * _NS

_NCOLS = (_SPK_NUM + 127) // 128
_COLS_PER_W = (_NCOLS + _NW - 1) // _NW
_CC = 2
_CW = _CC * 128
_NBUF = 4
_LCAP = 2048
_BCAP = 1024
_SENTINEL = 0x7FFFFFFF


def _rsqrt_vec(x):
    i = lax.bitcast_convert_type(x, jnp.int32)
    i = jnp.int32(0x5F3759DF) - lax.shift_right_arithmetic(i, 1)
    y = lax.bitcast_convert_type(i, jnp.float32)
    for _ in range(3):
        y = y * (1.5 - 0.5 * x * y * y)
    return y


_mesh = plsc.VectorSubcoreMesh(core_axis_name="c", subcore_axis_name="s")


@functools.partial(
    pl.kernel,
    out_type=jax.ShapeDtypeStruct((_BATCH + _L, 128), jnp.float32),
    mesh=_mesh,
    scratch_types=[
        pltpu.VMEM((_BATCH,), jnp.int32),
        pltpu.VMEM((_LCAP + _L,), jnp.int32),
        pltpu.VMEM((_LCAP + _L,), jnp.int32),
        pltpu.VMEM((_NBUF, _EMB_DIM, _CW), jnp.float32),
        pltpu.VMEM((_BCAP + _L,), jnp.int32),
        pltpu.VMEM((_BCAP + _L,), jnp.int32),
        pltpu.VMEM((_L, 128), jnp.float32),
        pltpu.SemaphoreType.DMA((_NBUF,)),
        pltpu.SemaphoreType.DMA,
    ],
    compiler_params=pltpu.CompilerParams(needs_layout_passes=False),
)
def _embed_normalize(idx_hbm, tableT_hbm, outP_hbm,
                     ids_v, fid_v, fpos_v, chunk_v, bco_v, bpos_v, rowbuf_v,
                     csem, rsem):
    wid = lax.axis_index("s") * _NC + lax.axis_index("c")
    lo_col = wid * _COLS_PER_W
    n_col = jnp.minimum(_COLS_PER_W, _NCOLS - lo_col)
    lo_id = lo_col * 128
    hi_id = (lo_col + n_col) * 128
    n_chunk = (n_col + _CC - 1) // _CC

    lane = lax.iota(jnp.int32, _L)

    def fetch(ch, slot):
        base_col = jnp.minimum(lo_col + ch * _CC, _NCOLS - _CC)
        pltpu.async_copy(
            tableT_hbm.at[:, pl.ds(base_col * 128, _CW)],
            chunk_v.at[slot],
            csem.at[slot],
        )

    for p in range(_NBUF - 1):
        fetch(jnp.int32(p), jnp.int32(p))

    pltpu.sync_copy(idx_hbm, ids_v)

    def filt(g, k):
        idv = ids_v[pl.ds(g * _L, _L)]
        mask = jnp.logical_and(idv >= lo_id, idv < hi_id)
        posv = g * _L + lane
        ks = jnp.minimum(k, _LCAP)
        plsc.store_compressed(fid_v.at[pl.ds(ks, _L)], idv, mask=mask)
        plsc.store_compressed(fpos_v.at[pl.ds(ks, _L)], posv, mask=mask)
        return ks + plsc.all_reduce_population_count(mask)[0]

    nloc = lax.fori_loop(0, _BATCH // _L, filt, jnp.int32(0))
    nloc = jnp.minimum(nloc, _LCAP)
    fid_v[pl.ds(nloc, _L)] = jnp.full((_L,), _SENTINEL, jnp.int32)
    n_grp = lax.shift_right_logical(nloc + (_L - 1), 4)

    def chunk_step(ch, _):
        slot = jnp.bitwise_and(ch, _NBUF - 1)
        pltpu.make_async_copy(
            tableT_hbm.at[:, pl.ds(0, _CW)], chunk_v.at[slot], csem.at[slot]
        ).wait()

        @pl.when(ch + (_NBUF - 1) < n_chunk)
        def _():
            fetch(ch + (_NBUF - 1),
                  jnp.bitwise_and(ch + (_NBUF - 1), _NBUF - 1))

        base_col = jnp.minimum(lo_col + ch * _CC, _NCOLS - _CC)
        cbase = base_col * 128
        wlo = jnp.maximum(jnp.maximum(cbase, lo_id), lo_id + ch * _CW)
        whi = jnp.minimum(cbase + _CW, hi_id)

        def scan(m, bk):
            idv = fid_v[pl.ds(m * _L, _L)]
            mask = jnp.logical_and(idv >= wlo, idv < whi)

            def hit(b):
                bs = jnp.minimum(b, _BCAP)
                plsc.store_compressed(
                    bco_v.at[pl.ds(bs, _L)], idv - cbase, mask=mask)
                plsc.store_compressed(
                    bpos_v.at[pl.ds(bs, _L)],
                    fpos_v[pl.ds(m * _L, _L)], mask=mask)
                return bs + plsc.all_reduce_population_count(mask)[0]

            any_hit = plsc.all_reduce_population_count(mask)[0] > 0
            return lax.cond(any_hit, hit, lambda b: b, bk)

        bk = lax.fori_loop(0, n_grp, scan, jnp.int32(0))
        bk = jnp.minimum(bk, _BCAP)

        def batch(q, _):
            coff = bco_v[pl.ds(q * _L, _L)]
            posb = bpos_v[pl.ds(q * _L, _L)]
            valid = lane < (bk - q * _L)
            coff = jnp.where(valid, coff, 0)
            posb = jnp.where(valid, posb, _BATCH + lane)

            acc = jnp.zeros((_L,), jnp.float32)
            for c in range(_EMB_DIM):
                crow = jnp.full((_L,), c, jnp.int32)
                v = plsc.load_gather(chunk_v.at[slot], [crow, coff])
                acc = acc + v * v
            inv = jnp.where(acc > 1e-24, _rsqrt_vec(acc), 1e12)

            for c in range(_EMB_DIM):
                crow = jnp.full((_L,), c, jnp.int32)
                v = plsc.load_gather(chunk_v.at[slot], [crow, coff])
                plsc.store_scatter(rowbuf_v, [lane, crow], v * inv)

            pltpu.async_copy(rowbuf_v, outP_hbm.at[posb], rsem).wait()
            return 0

        n_batch = lax.shift_right_logical(bk + (_L - 1), 4)
        lax.fori_loop(0, n_batch, batch, 0)
        return 0

    lax.fori_loop(0, n_chunk, chunk_step, 0)


def kernel(spk_ids, table):
    padded = _embed_normalize(spk_ids, table.T)
    return padded[:_BATCH, :_EMB_DIM]


# speedup vs baseline: 1.4417x; 1.4417x over previous
import functools

import jax
import jax.numpy as jnp
from jax import lax
from jax.experimental import pallas as pl
from jax.experimental.pallas import tpu as pltpu
from jax.experimental.pallas import tpu_sc as plsc

_SPK_NUM = 1000000
_EMB_DIM = 64
_BATCH = 16384

_NC = 2
_NS = 16
_L = 16
_NW = _NC * _NS

_NCOLS = (_SPK_NUM + 127) // 128
_COLS_PER_W = (_NCOLS + _NW - 1) // _NW
_CC = 4
_CW = _CC * 128
_NBUF = 3
_LCAP = 2048
_BCAP = 1024
_SENTINEL = 0x7FFFFFFF


def _rsqrt_vec(x):
    i = lax.bitcast_convert_type(x, jnp.int32)
    i = jnp.int32(0x5F3759DF) - lax.shift_right_arithmetic(i, 1)
    y = lax.bitcast_convert_type(i, jnp.float32)
    for _ in range(3):
        y = y * (1.5 - 0.5 * x * y * y)
    return y


_mesh = plsc.VectorSubcoreMesh(core_axis_name="c", subcore_axis_name="s")


@functools.partial(
    pl.kernel,
    out_type=jax.ShapeDtypeStruct((_BATCH + _L, 128), jnp.float32),
    mesh=_mesh,
    scratch_types=[
        pltpu.VMEM((_BATCH,), jnp.int32),
        pltpu.VMEM((_LCAP + _L,), jnp.int32),
        pltpu.VMEM((_LCAP + _L,), jnp.int32),
        pltpu.VMEM((_NBUF, _EMB_DIM, _CW), jnp.float32),
        pltpu.VMEM((_BCAP + _L,), jnp.int32),
        pltpu.VMEM((_BCAP + _L,), jnp.int32),
        pltpu.VMEM((2, _L, 128), jnp.float32),
        pltpu.SemaphoreType.DMA((_NBUF,)),
        pltpu.SemaphoreType.DMA((2,)),
    ],
    compiler_params=pltpu.CompilerParams(needs_layout_passes=False),
)
def _embed_normalize(idx_hbm, tableT_hbm, outP_hbm,
                     ids_v, fid_v, fpos_v, chunk_v, bco_v, bpos_v, rowbuf_v,
                     csem, rsem):
    wid = lax.axis_index("s") * _NC + lax.axis_index("c")
    lo_col = wid * _COLS_PER_W
    n_col = jnp.minimum(_COLS_PER_W, _NCOLS - lo_col)
    lo_id = lo_col * 128
    hi_id = (lo_col + n_col) * 128
    n_chunk = (n_col + _CC - 1) // _CC

    lane = lax.iota(jnp.int32, _L)

    def fetch(ch, slot):
        base_col = jnp.minimum(lo_col + ch * _CC, _NCOLS - _CC)
        pltpu.async_copy(
            tableT_hbm.at[:, pl.ds(base_col * 128, _CW)],
            chunk_v.at[slot],
            csem.at[slot],
        )

    for p in range(_NBUF - 1):
        fetch(jnp.int32(p), jnp.int32(p))

    pltpu.sync_copy(idx_hbm, ids_v)

    def filt(g, k):
        idv = ids_v[pl.ds(g * _L, _L)]
        mask = jnp.logical_and(idv >= lo_id, idv < hi_id)
        posv = g * _L + lane
        ks = jnp.minimum(k, _LCAP)
        plsc.store_compressed(fid_v.at[pl.ds(ks, _L)], idv, mask=mask)
        plsc.store_compressed(fpos_v.at[pl.ds(ks, _L)], posv, mask=mask)
        return ks + plsc.all_reduce_population_count(mask)[0]

    nloc = lax.fori_loop(0, _BATCH // _L, filt, jnp.int32(0))
    nloc = jnp.minimum(nloc, _LCAP)
    fid_v[pl.ds(nloc, _L)] = jnp.full((_L,), _SENTINEL, jnp.int32)
    n_grp = lax.shift_right_logical(nloc + (_L - 1), 4)

    def chunk_step(ch, gb):
        slot = lax.rem(ch, _NBUF)
        pltpu.make_async_copy(
            tableT_hbm.at[:, pl.ds(0, _CW)], chunk_v.at[slot], csem.at[slot]
        ).wait()

        @pl.when(ch + (_NBUF - 1) < n_chunk)
        def _():
            fetch(ch + (_NBUF - 1), lax.rem(ch + (_NBUF - 1), _NBUF))

        base_col = jnp.minimum(lo_col + ch * _CC, _NCOLS - _CC)
        cbase = base_col * 128
        wlo = jnp.maximum(jnp.maximum(cbase, lo_id), lo_id + ch * _CW)
        whi = jnp.minimum(cbase + _CW, hi_id)

        def scan(m, bk):
            idv = fid_v[pl.ds(m * _L, _L)]
            mask = jnp.logical_and(idv >= wlo, idv < whi)

            def hit(b):
                bs = jnp.minimum(b, _BCAP)
                plsc.store_compressed(
                    bco_v.at[pl.ds(bs, _L)], idv - cbase, mask=mask)
                plsc.store_compressed(
                    bpos_v.at[pl.ds(bs, _L)],
                    fpos_v[pl.ds(m * _L, _L)], mask=mask)
                return bs + plsc.all_reduce_population_count(mask)[0]

            any_hit = plsc.all_reduce_population_count(mask)[0] > 0
            return lax.cond(any_hit, hit, lambda b: b, bk)

        bk = lax.fori_loop(0, n_grp, scan, jnp.int32(0))
        bk = jnp.minimum(bk, _BCAP)

        def batch(q, gb):
            coff = bco_v[pl.ds(q * _L, _L)]
            posb = bpos_v[pl.ds(q * _L, _L)]
            valid = lane < (bk - q * _L)
            coff = jnp.where(valid, coff, 0)
            posb = jnp.where(valid, posb, _BATCH + lane)

            acc = jnp.zeros((_L,), jnp.float32)
            for c in range(_EMB_DIM):
                crow = jnp.full((_L,), c, jnp.int32)
                v = plsc.load_gather(chunk_v.at[slot], [crow, coff])
                acc = acc + v * v
            inv = jnp.where(acc > 1e-24, _rsqrt_vec(acc), 1e12)

            rslot = jnp.bitwise_and(gb, 1)

            @pl.when(gb >= 2)
            def _():
                pltpu.make_async_copy(
                    rowbuf_v.at[rslot], outP_hbm.at[pl.ds(0, _L)],
                    rsem.at[rslot],
                ).wait()

            for c in range(_EMB_DIM):
                crow = jnp.full((_L,), c, jnp.int32)
                v = plsc.load_gather(chunk_v.at[slot], [crow, coff])
                plsc.store_scatter(rowbuf_v.at[rslot], [lane, crow], v * inv)

            pltpu.async_copy(rowbuf_v.at[rslot], outP_hbm.at[posb],
                             rsem.at[rslot])
            return gb + 1

        n_batch = lax.shift_right_logical(bk + (_L - 1), 4)
        return lax.fori_loop(0, n_batch, batch, gb)

    gb = lax.fori_loop(0, n_chunk, chunk_step, jnp.int32(0))

    for s in range(2):
        @pl.when(gb > s)
        def _():
            pltpu.make_async_copy(
                rowbuf_v.at[s], outP_hbm.at[pl.ds(0, _L)], rsem.at[s]
            ).wait()


def kernel(spk_ids, table):
    padded = _embed_normalize(spk_ids, table.T)
    return padded[:_BATCH, :_EMB_DIM]


# speedup vs baseline: 2.0009x; 1.3879x over previous
import functools

import jax
import jax.numpy as jnp
from jax import lax
from jax.experimental import pallas as pl
from jax.experimental.pallas import tpu as pltpu
from jax.experimental.pallas import tpu_sc as plsc

_SPK_NUM = 1000000
_EMB_DIM = 64
_BATCH = 16384

_NC = 2
_NS = 16
_L = 16
_NW = _NC * _NS
_B_PER_W = _BATCH // _NW
_NG = _B_PER_W // _L
_K = 8


def _rsqrt(x):
    i = lax.bitcast_convert_type(x, jnp.int32)
    i = jnp.int32(0x5F3759DF) - lax.shift_right_arithmetic(i, 1)
    y = lax.bitcast_convert_type(i, jnp.float32)
    for _ in range(3):
        y = y * (1.5 - 0.5 * x * y * y)
    return y


_mesh = plsc.VectorSubcoreMesh(core_axis_name="c", subcore_axis_name="s")


@functools.partial(
    pl.kernel,
    out_type=jax.ShapeDtypeStruct((_EMB_DIM, _BATCH), jnp.float32),
    mesh=_mesh,
    scratch_types=[
        pltpu.VMEM((_B_PER_W,), jnp.int32),
        pltpu.VMEM((_K, _EMB_DIM, 128), jnp.float32),
        pltpu.VMEM((_EMB_DIM, _B_PER_W), jnp.float32),
        pltpu.SemaphoreType.DMA((_K,)),
    ],
    compiler_params=pltpu.CompilerParams(needs_layout_passes=False),
)
def _embed_normalize(idx_hbm, tableT_hbm, outT_hbm, idx_v, ring_v, cols_v, sems):
    wid = lax.axis_index("s") * _NC + lax.axis_index("c")
    base = wid * _B_PER_W

    pltpu.sync_copy(idx_hbm.at[pl.ds(base, _B_PER_W)], idx_v)

    lane = lax.iota(jnp.int32, _L)

    def fetch(i_id, slot):
        blk = lax.shift_right_logical(i_id, 7) * 128
        pltpu.async_copy(
            tableT_hbm.at[:, pl.ds(blk, 128)], ring_v.at[slot], sems.at[slot]
        )

    vec0 = idx_v[pl.ds(0, _L)]
    for j in range(_K):
        fetch(vec0[j], j)

    def group(g, _):
        vec = idx_v[pl.ds(g * _L, _L)]
        nxt_base = jnp.minimum((g + 1) * _L, _B_PER_W - _L)
        nxt = idx_v[pl.ds(nxt_base, _L)]

        for j in range(_L):
            i = g * _L + j
            slot = j % _K
            pltpu.make_async_copy(
                tableT_hbm.at[:, pl.ds(0, 128)], ring_v.at[slot], sems.at[slot]
            ).wait()

            m = jnp.bitwise_and(vec[j], 127)
            mcol = jnp.full((_L,), m, jnp.int32)
            icol = jnp.full((_L,), i, jnp.int32)
            for k in range(_EMB_DIM // _L):
                crow = lane + k * _L
                v = plsc.load_gather(ring_v.at[slot], [crow, mcol])
                plsc.store_scatter(cols_v, [crow, icol], v)

            if j < _L - _K:
                fetch(vec[j + _K], slot)
            else:
                @pl.when(g < _NG - 1)
                def _():
                    fetch(nxt[j + _K - _L], slot)
        return 0

    lax.fori_loop(0, _NG, group, 0)

    def norm_block(g, _):
        s = g * _L

        acc = jnp.zeros((_L,), jnp.float32)
        for c in range(_EMB_DIM):
            v = cols_v[c, pl.ds(s, _L)]
            acc = acc + v * v

        inv = jnp.where(acc > 1e-24, _rsqrt(acc), 1e12)

        for c in range(_EMB_DIM):
            cols_v[c, pl.ds(s, _L)] = cols_v[c, pl.ds(s, _L)] * inv
        return 0

    lax.fori_loop(0, _NG, norm_block, 0)

    pltpu.sync_copy(cols_v, outT_hbm.at[:, pl.ds(base, _B_PER_W)])


def kernel(spk_ids, table):
    outT = _embed_normalize(spk_ids, table.T)
    return outT.T
